# baseline (speedup 1.0000x reference)
import functools

import jax
import jax.numpy as jnp
from jax import lax
from jax.experimental import pallas as pl
from jax.experimental.pallas import tpu as pltpu
from jax.experimental.pallas import tpu_sc as plsc

B = 4096
S = 200
H = 64
HALF = S // 2


def _make_kernel():
    info = plsc.get_sparse_core_info()
    nc, ns = info.num_cores, info.num_subcores
    nw = nc * ns
    b_per_w = B // nw

    mesh = plsc.VectorSubcoreMesh(core_axis_name="c", subcore_axis_name="s")

    @functools.partial(
        pl.kernel,
        mesh=mesh,
        out_type=jax.ShapeDtypeStruct((B, S, H), jnp.float32),
        compiler_params=pltpu.CompilerParams(use_tc_tiling_on_sc=False),
        scratch_types=[
            pltpu.VMEM((b_per_w,), jnp.int32),
            pltpu.VMEM((b_per_w, H), jnp.float32),
            pltpu.VMEM((b_per_w, 2, HALF), jnp.int32),
            pltpu.VMEM((S, H), jnp.float32),
            pltpu.VMEM((S, H), jnp.float32),
            pltpu.VMEM((S, H), jnp.float32),
            pltpu.VMEM((S, H), jnp.float32),
            pltpu.SemaphoreType.DMA,
            pltpu.SemaphoreType.DMA,
            pltpu.SemaphoreType.DMA,
            pltpu.SemaphoreType.DMA,
            pltpu.SemaphoreType.DMA,
            pltpu.SemaphoreType.DMA,
        ],
    )
    def k(hid_hbm, iids_hbm, sids_hbm, itab_hbm, stab_hbm, out_hbm,
          iid_v, irows_v, sid_v, hid0, hid1, srows0, srows1,
          hsem0, hsem1, gsem0, gsem1, osem0, osem1):
        wid = lax.axis_index("s") * nc + lax.axis_index("c")
        b0 = wid * b_per_w

        bufs = ((hid0, srows0, hsem0, gsem0, osem0),
                (hid1, srows1, hsem1, gsem1, osem1))

        pltpu.sync_copy(iids_hbm.at[pl.ds(b0, b_per_w)], iid_v)
        pltpu.sync_copy(sids_hbm.at[pl.ds(b0, b_per_w)], sid_v)
        pltpu.async_copy(itab_hbm.at[iid_v], irows_v, gsem0).wait()

        def issue_in(m, hid, srows, hsem, gsem):
            pltpu.async_copy(hid_hbm.at[b0 + m], hid, hsem)
            pltpu.async_copy(
                stab_hbm.at[sid_v.at[m, 0]], srows.at[pl.ds(0, HALF)], gsem)
            pltpu.async_copy(
                stab_hbm.at[sid_v.at[m, 1]], srows.at[pl.ds(HALF, HALF)], gsem)

        def wait_in(hid, srows, hsem, gsem):
            pltpu.make_async_copy(hid_hbm.at[0], hid, hsem).wait()
            pltpu.make_async_copy(
                stab_hbm.at[pl.ds(0, HALF)], srows.at[pl.ds(0, HALF)], gsem).wait()
            pltpu.make_async_copy(
                stab_hbm.at[pl.ds(0, HALF)], srows.at[pl.ds(HALF, HALF)], gsem).wait()

        issue_in(0, hid0, srows0, hsem0, gsem0)

        def pair_body(g, _):
            for j in (0, 1):
                hid, srows, hsem, gsem, osem = bufs[j]
                nhid, nsrows, nhsem, ngsem, nosem = bufs[1 - j]
                cur = 2 * g + j

                @pl.when(cur >= 1)
                def _():
                    pltpu.make_async_copy(
                        nhid, out_hbm.at[b0], nosem).wait()

                @pl.when(cur + 1 < b_per_w)
                def _():
                    issue_in(cur + 1, nhid, nsrows, nhsem, ngsem)

                wait_in(hid, srows, hsem, gsem)

                iv0 = irows_v[cur, pl.ds(0, 16)]
                iv1 = irows_v[cur, pl.ds(16, 16)]
                iv2 = irows_v[cur, pl.ds(32, 16)]
                iv3 = irows_v[cur, pl.ds(48, 16)]

                ivs = (iv0, iv1, iv2, iv3)

                @plsc.parallel_loop(0, S, step=1, unroll=4)
                def row_body(r):
                    for kk in range(4):
                        sl = pl.ds(16 * kk, 16)
                        plsc.addupdate(hid.at[r, sl], srows[r, sl] + ivs[kk])
                pltpu.async_copy(hid, out_hbm.at[b0 + cur], osem)
            return 0

        lax.fori_loop(0, b_per_w // 2, pair_body, 0)
        pltpu.make_async_copy(hid1, out_hbm.at[b0], osem1).wait()

    return k


_kernel_call = None


def kernel(hidden_states, instrument_ids, session_ids, instrument_table, session_table):
    global _kernel_call
    if _kernel_call is None:
        _kernel_call = _make_kernel()
    sids = session_ids.reshape(B, 2, HALF).astype(jnp.int32)
    iids = instrument_ids.astype(jnp.int32)
    return _kernel_call(hidden_states, iids, sids, instrument_table, session_table)


# speedup vs baseline: 1.1058x; 1.1058x over previous
import functools

import jax
import jax.numpy as jnp
from jax import lax
from jax.experimental import pallas as pl
from jax.experimental.pallas import tpu as pltpu
from jax.experimental.pallas import tpu_sc as plsc

B = 4096
S = 200
H = 64
ROWS_PER_BATCH = S * H // 128
GROUP = 16
GROUP_ROWS = GROUP * S // 128


def _make_kernel():
    info = plsc.get_sparse_core_info()
    nc, ns = info.num_cores, info.num_subcores
    nw = nc * ns
    b_per_w = B // nw

    mesh = plsc.VectorSubcoreMesh(core_axis_name="c", subcore_axis_name="s")

    @functools.partial(
        pl.kernel,
        mesh=mesh,
        out_type=jax.ShapeDtypeStruct((B * S * H // 128, 128), jnp.float32),
        compiler_params=pltpu.CompilerParams(
            use_tc_tiling_on_sc=False, needs_layout_passes=False),
        scratch_types=[
            pltpu.VMEM((b_per_w,), jnp.int32),
            pltpu.VMEM((b_per_w,), jnp.int32),
            pltpu.VMEM((b_per_w, 128), jnp.float32),
            pltpu.VMEM((1000 * H // 128, 128), jnp.float32),
            pltpu.VMEM((GROUP_ROWS, 128), jnp.int32),
            pltpu.VMEM((ROWS_PER_BATCH, 128), jnp.float32),
            pltpu.VMEM((ROWS_PER_BATCH, 128), jnp.float32),
            pltpu.SemaphoreType.DMA,
            pltpu.SemaphoreType.DMA,
            pltpu.SemaphoreType.DMA,
            pltpu.SemaphoreType.DMA,
            pltpu.SemaphoreType.DMA,
        ],
    )
    def k(hid_hbm, iids_hbm, sids_hbm, itab_hbm, stab_hbm, out_hbm,
          iid_v, iidh_v, irowsp_v, table_v, sidg_v, hid0, hid1,
          hsem0, hsem1, osem0, osem1, gsem):
        wid = lax.axis_index("s") * nc + lax.axis_index("c")
        b0 = wid * b_per_w
        iota = lax.iota(jnp.int32, 16)

        bufs = ((hid0, hsem0, osem0), (hid1, hsem1, osem1))

        pltpu.sync_copy(stab_hbm, table_v)
        pltpu.sync_copy(iids_hbm.at[pl.ds(b0, b_per_w)], iid_v)
        for i in range(b_per_w // 16):
            iidh_v[pl.ds(16 * i, 16)] = iid_v[pl.ds(16 * i, 16)] >> 1
        pltpu.async_copy(itab_hbm.at[iidh_v], irowsp_v, gsem).wait()
        pltpu.sync_copy(
            sids_hbm.at[pl.ds(wid * (b_per_w * S // 128), GROUP_ROWS)], sidg_v)

        def issue_in(m, hid, hsem):
            pltpu.async_copy(
                hid_hbm.at[pl.ds((b0 + m) * ROWS_PER_BATCH, ROWS_PER_BATCH)],
                hid, hsem)

        issue_in(0, hid0, hsem0)

        def compute(m, hid):
            mv = jnp.full((16,), m, jnp.int32)
            bidv = plsc.load_gather(iid_v, [mv])
            par = (bidv & 1) * 64
            ivs = []
            for kk in range(4):
                ivs.append(plsc.load_gather(
                    irowsp_v, [mv, par + (16 * kk + iota)]))
            gbase = lax.rem(m, GROUP) * S

            @plsc.parallel_loop(0, ROWS_PER_BATCH, step=1, unroll=2)
            def p_body(p):
                f0 = gbase + 2 * p
                for h in (0, 1):
                    f = f0 + h
                    idv = plsc.load_gather(
                        sidg_v,
                        [jnp.full((16,), f >> 7, jnp.int32),
                         jnp.full((16,), f & 127, jnp.int32)])
                    base = idv * 64
                    for kk in range(4):
                        addr = base + (16 * kk + iota)
                        srow = plsc.load_gather(
                            table_v, [addr >> 7, addr & 127])
                        plsc.addupdate(
                            hid.at[p, pl.ds(h * 64 + 16 * kk, 16)],
                            srow + ivs[kk])

        def pair_body(g, _):
            @pl.when(jnp.logical_and(lax.rem(g, 8) == 0, g > 0))
            def _():
                pltpu.sync_copy(
                    sids_hbm.at[pl.ds(
                        wid * (b_per_w * S // 128) + (g // 8) * GROUP_ROWS,
                        GROUP_ROWS)],
                    sidg_v)

            for j in (0, 1):
                hid, hsem, osem = bufs[j]
                nhid, nhsem, nosem = bufs[1 - j]
                cur = 2 * g + j

                @pl.when(cur >= 1)
                def _():
                    pltpu.make_async_copy(
                        nhid, out_hbm.at[pl.ds(0, ROWS_PER_BATCH)], nosem).wait()

                @pl.when(cur + 1 < b_per_w)
                def _():
                    issue_in(cur + 1, nhid, nhsem)

                pltpu.make_async_copy(
                    hid_hbm.at[pl.ds(0, ROWS_PER_BATCH)], hid, hsem).wait()
                compute(cur, hid)
                pltpu.async_copy(
                    hid,
                    out_hbm.at[pl.ds((b0 + cur) * ROWS_PER_BATCH, ROWS_PER_BATCH)],
                    osem)
            return 0

        lax.fori_loop(0, b_per_w // 2, pair_body, 0)
        pltpu.make_async_copy(
            hid1, out_hbm.at[pl.ds(0, ROWS_PER_BATCH)], osem1).wait()

    return k


_kernel_call = None


def kernel(hidden_states, instrument_ids, session_ids, instrument_table, session_table):
    global _kernel_call
    if _kernel_call is None:
        _kernel_call = _make_kernel()
    hid2 = hidden_states.reshape(-1, 128)
    sids2 = session_ids.astype(jnp.int32).reshape(-1, 128)
    iids = instrument_ids.astype(jnp.int32)
    itab2 = instrument_table.reshape(-1, 128)
    stab2 = session_table.reshape(-1, 128)
    out2 = _kernel_call(hid2, iids, sids2, itab2, stab2)
    return out2.reshape(B, S, H)


# speedup vs baseline: 1.9330x; 1.7480x over previous
import functools

import jax
import jax.numpy as jnp
from jax import lax
from jax.experimental import pallas as pl
from jax.experimental.pallas import tpu as pltpu
from jax.experimental.pallas import tpu_sc as plsc

B = 4096
S = 200
H = 64
NUM_INST = 100000
NUM_SESS = 1000


def _make_inst_kernel():
    info = plsc.get_sparse_core_info()
    nc, ns = info.num_cores, info.num_subcores
    nw = nc * ns
    rows_per_w = H // nw

    mesh = plsc.VectorSubcoreMesh(core_axis_name="c", subcore_axis_name="s")

    @functools.partial(
        pl.kernel,
        mesh=mesh,
        out_type=jax.ShapeDtypeStruct((H, B), jnp.float32),
        compiler_params=pltpu.CompilerParams(
            use_tc_tiling_on_sc=False, needs_layout_passes=False),
        scratch_types=[
            pltpu.VMEM((NUM_INST,), jnp.float32),
            pltpu.VMEM((B,), jnp.int32),
            pltpu.VMEM((B,), jnp.float32),
        ],
    )
    def k(itab_hbm, iids_hbm, iemb_hbm, row_v, iid_v, orow_v):
        wid = lax.axis_index("s") * nc + lax.axis_index("c")
        pltpu.sync_copy(iids_hbm, iid_v)
        for t in range(rows_per_w):
            h = wid * rows_per_w + t
            pltpu.sync_copy(itab_hbm.at[h], row_v)

            @plsc.parallel_loop(0, B // 16, step=1, unroll=4)
            def c_body(c):
                idxv = iid_v[pl.ds(16 * c, 16)]
                orow_v[pl.ds(16 * c, 16)] = plsc.load_gather(row_v, [idxv])

            pltpu.sync_copy(orow_v, iemb_hbm.at[h])

    return k


def _make_main_kernel():
    info = plsc.get_sparse_core_info()
    nc, ns = info.num_cores, info.num_subcores
    nw = nc * ns
    cols_per_w = B // nw

    mesh = plsc.VectorSubcoreMesh(core_axis_name="c", subcore_axis_name="s")

    @functools.partial(
        pl.kernel,
        mesh=mesh,
        out_type=jax.ShapeDtypeStruct((S, H, B), jnp.float32),
        compiler_params=pltpu.CompilerParams(
            use_tc_tiling_on_sc=False, needs_layout_passes=False),
        scratch_types=[
            pltpu.VMEM((H, NUM_SESS), jnp.float32),
            pltpu.VMEM((H, 128), jnp.float32),
            pltpu.VMEM((S, 128), jnp.int32),
            pltpu.VMEM((H, 128), jnp.float32),
            pltpu.VMEM((H, 128), jnp.float32),
            pltpu.SemaphoreType.DMA,
            pltpu.SemaphoreType.DMA,
            pltpu.SemaphoreType.DMA,
            pltpu.SemaphoreType.DMA,
        ],
    )
    def k(hid_hbm, sid_hbm, stab_hbm, iemb_hbm, out_hbm,
          table_v, iemb_v, sid_v, hid0, hid1, hsem0, hsem1, osem0, osem1):
        wid = lax.axis_index("s") * nc + lax.axis_index("c")
        c0 = wid * cols_per_w

        bufs = ((hid0, hsem0, osem0), (hid1, hsem1, osem1))

        pltpu.sync_copy(stab_hbm, table_v)
        pltpu.sync_copy(iemb_hbm.at[:, pl.ds(c0, 128)], iemb_v)
        pltpu.sync_copy(sid_hbm.at[:, pl.ds(c0, 128)], sid_v)

        def issue_in(s, hid, hsem):
            pltpu.async_copy(hid_hbm.at[s, :, pl.ds(c0, 128)], hid, hsem)

        issue_in(0, hid0, hsem0)

        def compute(s, hid):
            idxs = [sid_v[s, pl.ds(16 * c, 16)] for c in range(8)]

            @plsc.parallel_loop(0, H, step=1, unroll=2)
            def h_body(h):
                hv = jnp.full((16,), h, jnp.int32)
                for c in range(8):
                    srow = plsc.load_gather(table_v, [hv, idxs[c]])
                    plsc.addupdate(
                        hid.at[h, pl.ds(16 * c, 16)],
                        srow + iemb_v[h, pl.ds(16 * c, 16)])

        def pair_body(g, _):
            for j in (0, 1):
                hid, hsem, osem = bufs[j]
                nhid, nhsem, nosem = bufs[1 - j]
                s = 2 * g + j

                @pl.when(s >= 1)
                def _():
                    pltpu.make_async_copy(
                        nhid, out_hbm.at[0, :, pl.ds(0, 128)], nosem).wait()

                @pl.when(s + 1 < S)
                def _():
                    issue_in(s + 1, nhid, nhsem)

                pltpu.make_async_copy(
                    hid_hbm.at[0, :, pl.ds(0, 128)], hid, hsem).wait()
                compute(s, hid)
                pltpu.async_copy(
                    hid, out_hbm.at[s, :, pl.ds(c0, 128)], osem)
            return 0

        lax.fori_loop(0, S // 2, pair_body, 0)
        pltpu.make_async_copy(
            hid1, out_hbm.at[0, :, pl.ds(0, 128)], osem1).wait()

    return k


_inst_call = None
_main_call = None


def kernel(hidden_states, instrument_ids, session_ids, instrument_table, session_table):
    global _inst_call, _main_call
    if _inst_call is None:
        _inst_call = _make_inst_kernel()
        _main_call = _make_main_kernel()
    hid_t = jnp.transpose(hidden_states, (1, 2, 0))
    sid_t = jnp.transpose(session_ids.astype(jnp.int32), (1, 0))
    itab_t = jnp.transpose(instrument_table, (1, 0))
    stab_t = jnp.transpose(session_table, (1, 0))
    iids = instrument_ids.astype(jnp.int32)
    iemb = _inst_call(itab_t, iids)
    out_t = _main_call(hid_t, sid_t, stab_t, iemb)
    return jnp.transpose(out_t, (2, 0, 1))
